# baseline (speedup 1.0000x reference)
import functools
import math

import jax
import jax.numpy as jnp
from jax import lax
from jax.experimental import pallas as pl
from jax.experimental.pallas import tpu as pltpu
from jax.experimental.pallas import tpu_sc as plsc

_NUM_LEVELS = 12
_BASE_RES = 16
_DESIRED_RES = 8192
_HASHMAP_SIZE = 2 ** 21
_PER_LEVEL_SCALE = 2.0 ** (math.log2(_DESIRED_RES / _BASE_RES) / (_NUM_LEVELS - 1))
_PRIMES = (1, 2654435761, 805459861)

_F32 = jnp.float32
_I32 = jnp.int32
_NW = 32


def _level_meta(input_dim):
    metas = []
    offset = 0
    for l in range(_NUM_LEVELS):
        scale = _BASE_RES * (_PER_LEVEL_SCALE ** l) - 1.0
        res = int(math.ceil(scale)) + 1
        params = min(_HASHMAP_SIZE, (res + 1) ** input_dim)
        params = int(math.ceil(params / 8) * 8)
        dense = (res + 1) ** input_dim <= _HASHMAP_SIZE
        metas.append((scale, res, params, offset, dense))
        offset += params
    return metas


def _as_i32(v):
    v = v & 0xFFFFFFFF
    return v - (1 << 32) if v >= (1 << 31) else v


def _make_encode(dim, n_pts, chunk):
    metas = _level_meta(dim)
    ncorner = 2 ** dim
    npw = n_pts // _NW
    nch = npw // chunk
    C = chunk
    grp = C // 16
    mesh = plsc.VectorSubcoreMesh(core_axis_name="c", subcore_axis_name="s")

    @functools.partial(
        pl.kernel,
        mesh=mesh,
        compiler_params=pltpu.CompilerParams(
            needs_layout_passes=False, use_tc_tiling_on_sc=False),
        out_type=jax.ShapeDtypeStruct((2 * _NUM_LEVELS, n_pts), _F32),
        scratch_types=(
            [pltpu.VMEM((C,), _F32) for _ in range(dim)]
            + [
                pltpu.VMEM((2 * ncorner, C), _I32),
                pltpu.VMEM((ncorner, C), _F32),
                pltpu.VMEM((2 * ncorner, C), _F32),
                pltpu.VMEM((2 * _NUM_LEVELS, C), _F32),
                pltpu.SemaphoreType.DMA,
            ]
        ),
    )
    def enc(*refs):
        coords = refs[:dim]
        table = refs[dim]
        out = refs[dim + 1]
        ubs = refs[dim + 2 : 2 * dim + 2]
        idxb, wb, gb, featsb, sem = refs[2 * dim + 2 :]

        wid = lax.axis_index("s") * 2 + lax.axis_index("c")
        base = wid * npw

        def chunk_body(ch, carry):
            p0 = base + ch * C
            for d in range(dim):
                pltpu.sync_copy(coords[d].at[pl.ds(p0, C)], ubs[d])

            def u_body(g, c2):
                s = g * 16
                for d in range(dim):
                    x = ubs[d][pl.ds(s, 16)]
                    ubs[d][pl.ds(s, 16)] = (x + 1.0) * 0.5
                return c2

            lax.fori_loop(0, grp, u_body, 0)

            for lvl, (scale, res, params, offset, dense) in enumerate(metas):

                def pa_body(g, c2, scale=scale, res=res, params=params,
                            offset=offset, dense=dense):
                    s = g * 16
                    pis = []
                    fr = []
                    for d in range(dim):
                        p = ubs[d][pl.ds(s, 16)] * _F32(scale) + _F32(0.5)
                        pi = p.astype(_I32)
                        pis.append(pi)
                        fr.append(p - pi.astype(_F32))
                    a = []
                    if dense:
                        strides = [1, res + 1, (res + 1) ** 2]
                        lo = pis[0] + offset
                        a.append((lo, lo + 1))
                        for d in range(1, dim):
                            lo = pis[d] * strides[d]
                            a.append((lo, lo + strides[d]))
                    else:
                        a.append((pis[0], pis[0] + 1))
                        for d in range(1, dim):
                            pr = _as_i32(_PRIMES[d])
                            lo = pis[d] * pr
                            a.append((lo, lo + pr))
                    w01 = [(_F32(1.0) - fr[d], fr[d]) for d in range(dim)]
                    for c in range(ncorner):
                        bits = [(c >> d) & 1 for d in range(dim)]
                        if dense:
                            idx = a[0][bits[0]]
                            for d in range(1, dim):
                                idx = idx + a[d][bits[d]]
                        else:
                            h = a[0][bits[0]]
                            for d in range(1, dim):
                                h = lax.bitwise_xor(h, a[d][bits[d]])
                            idx = lax.bitwise_and(h, params - 1) + offset
                        e0 = idx + idx
                        idxb[2 * c, pl.ds(s, 16)] = e0
                        idxb[2 * c + 1, pl.ds(s, 16)] = e0 + 1
                        w = w01[0][bits[0]]
                        for d in range(1, dim):
                            w = w * w01[d][bits[d]]
                        wb[c, pl.ds(s, 16)] = w
                    return c2

                lax.fori_loop(0, grp, pa_body, 0)

                cps = [
                    pltpu.async_copy(table.at[idxb.at[r]], gb.at[r], sem)
                    for r in range(2 * ncorner)
                ]
                for cp in cps:
                    cp.wait()

                def pc_body(g, c2, lvl=lvl):
                    s = g * 16
                    acc0 = jnp.zeros((16,), _F32)
                    acc1 = jnp.zeros((16,), _F32)
                    for c in range(ncorner):
                        wv = wb[c, pl.ds(s, 16)]
                        acc0 = acc0 + wv * gb[2 * c, pl.ds(s, 16)]
                        acc1 = acc1 + wv * gb[2 * c + 1, pl.ds(s, 16)]
                    featsb[2 * lvl, pl.ds(s, 16)] = acc0
                    featsb[2 * lvl + 1, pl.ds(s, 16)] = acc1
                    return c2

                lax.fori_loop(0, grp, pc_body, 0)

            pltpu.sync_copy(featsb, out.at[:, pl.ds(p0, C)])
            return carry

        lax.fori_loop(0, nch, chunk_body, 0)

    return enc


def _mlp_t(feats_t, W1, b1, W2, b2, pts_blk):
    fd, n = feats_t.shape
    grid = n // pts_blk
    dn1 = (((0,), (0,)), ((), ()))

    def body(f_ref, w1_ref, b1_ref, w2_ref, b2_ref, o_ref):
        f = f_ref[...]
        h = lax.dot_general(w1_ref[...], f, dn1,
                            preferred_element_type=_F32) + b1_ref[...]
        h = jnp.maximum(h, 0.0)
        o_ref[...] = lax.dot_general(w2_ref[...], h, dn1,
                                     preferred_element_type=_F32) + b2_ref[0, 0]

    return pl.pallas_call(
        body,
        grid=(grid,),
        in_specs=[
            pl.BlockSpec((fd, pts_blk), lambda i: (0, i)),
            pl.BlockSpec((fd, 64), lambda i: (0, 0)),
            pl.BlockSpec((64, 1), lambda i: (0, 0)),
            pl.BlockSpec((64, 1), lambda i: (0, 0)),
            pl.BlockSpec((1, 1), lambda i: (0, 0)),
        ],
        out_specs=pl.BlockSpec((1, pts_blk), lambda i: (0, i)),
        out_shape=jax.ShapeDtypeStruct((1, n), _F32),
    )(feats_t, W1, b1.reshape(64, 1), W2, b2.reshape(1, 1))


def _row_loss(op_bl, gt_o):
    B, L = gt_o.shape
    rb = 256
    half = L // 2

    def body(o_ref, g_ref, m_ref):
        d = jnp.abs(o_ref[...] - g_ref[...])
        col = lax.broadcasted_iota(_I32, (rb, L), 1)
        w = jnp.where(col < half, _F32(4.0 / 3.0), _F32(2.0 / 3.0))
        m_ref[...] = jnp.mean(d * w, axis=1, keepdims=True)

    return pl.pallas_call(
        body,
        grid=(B // rb,),
        in_specs=[
            pl.BlockSpec((rb, L), lambda i: (i, 0)),
            pl.BlockSpec((rb, L), lambda i: (i, 0)),
        ],
        out_specs=pl.BlockSpec((rb, 1), lambda i: (i, 0)),
        out_shape=jax.ShapeDtypeStruct((B, 1), _F32),
    )(op_bl, gt_o)


def _final_loss(cf_feats_t, m, W1, b1, W2, b2):
    fd, B = cf_feats_t.shape
    dn1 = (((0,), (0,)), ((), ()))

    def body(f_ref, m_ref, w1_ref, b1_ref, w2_ref, b2_ref, o_ref):
        h = lax.dot_general(w1_ref[...], f_ref[...], dn1,
                            preferred_element_type=_F32) + b1_ref[...]
        h = jnp.maximum(h, 0.0)
        conf = lax.dot_general(w2_ref[...], h, dn1,
                               preferred_element_type=_F32) + b2_ref[0, 0]
        lv = jnp.exp(-conf) * m_ref[...] + conf
        o_ref[...] = jnp.mean(lv).reshape(1, 1)

    return pl.pallas_call(
        body,
        in_specs=[
            pl.BlockSpec((fd, B), lambda: (0, 0)),
            pl.BlockSpec((1, B), lambda: (0, 0)),
            pl.BlockSpec((fd, 64), lambda: (0, 0)),
            pl.BlockSpec((64, 1), lambda: (0, 0)),
            pl.BlockSpec((64, 1), lambda: (0, 0)),
            pl.BlockSpec((1, 1), lambda: (0, 0)),
        ],
        out_specs=pl.BlockSpec((1, 1), lambda: (0, 0)),
        out_shape=jax.ShapeDtypeStruct((1, 1), _F32),
    )(cf_feats_t, m, W1, b1.reshape(64, 1), W2, b2.reshape(1, 1))


def kernel(line, line_points, gt_o, op_table, op_W1, op_b1, op_W2, op_b2,
           cf_table, cf_W1, cf_b1, cf_W2, cf_b2):
    B, L, _ = line_points.shape
    n = B * L

    pts = line_points.reshape(n, 3).T
    ln = line.T

    enc3 = _make_encode(3, n, 128)
    enc2 = _make_encode(2, B, B // _NW)
    op_feats = enc3(pts[0], pts[1], pts[2], op_table.reshape(-1))
    cf_feats = enc2(ln[0], ln[1], cf_table.reshape(-1))

    opacity = _mlp_t(op_feats, op_W1, op_b1, op_W2, op_b2, 16000)
    m = _row_loss(opacity.reshape(B, L), gt_o)
    out = _final_loss(cf_feats, m.reshape(1, B), cf_W1, cf_b1, cf_W2, cf_b2)
    return out.reshape(())


# speedup vs baseline: 1.1063x; 1.1063x over previous
import functools
import math

import jax
import jax.numpy as jnp
from jax import lax
from jax.experimental import pallas as pl
from jax.experimental.pallas import tpu as pltpu
from jax.experimental.pallas import tpu_sc as plsc

_NUM_LEVELS = 12
_BASE_RES = 16
_DESIRED_RES = 8192
_HASHMAP_SIZE = 2 ** 21
_PER_LEVEL_SCALE = 2.0 ** (math.log2(_DESIRED_RES / _BASE_RES) / (_NUM_LEVELS - 1))
_PRIMES = (1, 2654435761, 805459861)

_F32 = jnp.float32
_I32 = jnp.int32
_NW = 32


def _level_meta(input_dim):
    metas = []
    offset = 0
    for l in range(_NUM_LEVELS):
        scale = _BASE_RES * (_PER_LEVEL_SCALE ** l) - 1.0
        res = int(math.ceil(scale)) + 1
        params = min(_HASHMAP_SIZE, (res + 1) ** input_dim)
        params = int(math.ceil(params / 8) * 8)
        dense = (res + 1) ** input_dim <= _HASHMAP_SIZE
        metas.append((scale, res, params, offset, dense))
        offset += params
    return metas


def _as_i32(v):
    v = v & 0xFFFFFFFF
    return v - (1 << 32) if v >= (1 << 31) else v


def _make_encode(dim, n_pts, chunk):
    metas = _level_meta(dim)
    ncorner = 2 ** dim
    npw = n_pts // _NW
    nch = npw // chunk
    C = chunk
    grp = C // 16
    nel = 2 * ncorner * C
    krows = nel // 128
    assert nel % 128 == 0
    mesh = plsc.VectorSubcoreMesh(core_axis_name="c", subcore_axis_name="s")

    @functools.partial(
        pl.kernel,
        mesh=mesh,
        compiler_params=pltpu.CompilerParams(
            needs_layout_passes=False, use_tc_tiling_on_sc=False),
        out_type=jax.ShapeDtypeStruct((2 * _NUM_LEVELS, n_pts), _F32),
        scratch_types=(
            [pltpu.VMEM((C,), _F32) for _ in range(dim)]
            + [
                pltpu.VMEM((2, nel), _I32),
                pltpu.VMEM((2, ncorner, C), _F32),
                pltpu.VMEM((2, nel), _F32),
                pltpu.VMEM((2 * _NUM_LEVELS, C), _F32),
                pltpu.SemaphoreType.DMA,
                pltpu.SemaphoreType.DMA,
            ]
        ),
    )
    def enc(*refs):
        coords = refs[:dim]
        table = refs[dim]
        out = refs[dim + 1]
        ubs = refs[dim + 2 : 2 * dim + 2]
        idxb, wb, gb, featsb, sem0, sem1 = refs[2 * dim + 2 :]
        sems = (sem0, sem1)

        wid = lax.axis_index("s") * 2 + lax.axis_index("c")
        base = wid * npw

        def phase_a(buf, scale, res, params, offset, dense):
            def pa_body(g, c2):
                s = g * 16
                pis = []
                fr = []
                for d in range(dim):
                    p = ubs[d][pl.ds(s, 16)] * _F32(scale) + _F32(0.5)
                    pi = p.astype(_I32)
                    pis.append(pi)
                    fr.append(p - pi.astype(_F32))
                a = []
                if dense:
                    strides = [1, res + 1, (res + 1) ** 2]
                    lo = pis[0] + offset
                    a.append((lo, lo + 1))
                    for d in range(1, dim):
                        lo = pis[d] * strides[d]
                        a.append((lo, lo + strides[d]))
                else:
                    a.append((pis[0], pis[0] + 1))
                    for d in range(1, dim):
                        pr = _as_i32(_PRIMES[d])
                        lo = pis[d] * pr
                        a.append((lo, lo + pr))
                w01 = [(_F32(1.0) - fr[d], fr[d]) for d in range(dim)]
                for c in range(ncorner):
                    bits = [(c >> d) & 1 for d in range(dim)]
                    if dense:
                        idx = a[0][bits[0]]
                        for d in range(1, dim):
                            idx = idx + a[d][bits[d]]
                    else:
                        h = a[0][bits[0]]
                        for d in range(1, dim):
                            h = lax.bitwise_xor(h, a[d][bits[d]])
                        idx = lax.bitwise_and(h, params - 1) + offset
                    e0 = idx + idx
                    b0, b1 = 2 * c * C, 2 * c * C + C
                    idxb[buf, pl.ds(b0 + s, 16)] = e0
                    idxb[buf, pl.ds(b1 + s, 16)] = e0 + 1
                    w = w01[0][bits[0]]
                    for d in range(1, dim):
                        w = w * w01[d][bits[d]]
                    wb[buf, c, pl.ds(s, 16)] = w
                return c2

            lax.fori_loop(0, grp, pa_body, 0)

        def phase_c(buf, lvl):
            def pc_body(g, c2):
                s = g * 16
                acc0 = jnp.zeros((16,), _F32)
                acc1 = jnp.zeros((16,), _F32)
                for c in range(ncorner):
                    wv = wb[buf, c, pl.ds(s, 16)]
                    b0, b1 = 2 * c * C, 2 * c * C + C
                    acc0 = acc0 + wv * gb[buf, pl.ds(b0 + s, 16)]
                    acc1 = acc1 + wv * gb[buf, pl.ds(b1 + s, 16)]
                featsb[2 * lvl, pl.ds(s, 16)] = acc0
                featsb[2 * lvl + 1, pl.ds(s, 16)] = acc1
                return c2

            lax.fori_loop(0, grp, pc_body, 0)

        def chunk_body(ch, carry):
            p0 = base + ch * C
            for d in range(dim):
                pltpu.sync_copy(coords[d].at[pl.ds(p0, C)], ubs[d])

            def u_body(g, c2):
                s = g * 16
                for d in range(dim):
                    x = ubs[d][pl.ds(s, 16)]
                    ubs[d][pl.ds(s, 16)] = (x + 1.0) * 0.5
                return c2

            lax.fori_loop(0, grp, u_body, 0)

            cps = [None, None]
            for lvl, (scale, res, params, offset, dense) in enumerate(metas):
                buf = lvl % 2
                phase_a(buf, scale, res, params, offset, dense)
                cps[buf] = pltpu.async_copy(
                    table.at[idxb.at[buf]], gb.at[buf], sems[buf])
                if lvl > 0:
                    cps[1 - buf].wait()
                    phase_c(1 - buf, lvl - 1)
            cps[1].wait()
            phase_c(1, _NUM_LEVELS - 1)

            pltpu.sync_copy(featsb, out.at[:, pl.ds(p0, C)])
            return carry

        lax.fori_loop(0, nch, chunk_body, 0)

    return enc


def _mlp_t(feats_t, W1, b1, W2, b2, pts_blk):
    fd, n = feats_t.shape
    grid = n // pts_blk
    dn1 = (((0,), (0,)), ((), ()))

    def body(f_ref, w1_ref, b1_ref, w2_ref, b2_ref, o_ref):
        f = f_ref[...]
        h = lax.dot_general(w1_ref[...], f, dn1,
                            preferred_element_type=_F32) + b1_ref[...]
        h = jnp.maximum(h, 0.0)
        o_ref[...] = lax.dot_general(w2_ref[...], h, dn1,
                                     preferred_element_type=_F32) + b2_ref[0, 0]

    return pl.pallas_call(
        body,
        grid=(grid,),
        in_specs=[
            pl.BlockSpec((fd, pts_blk), lambda i: (0, i)),
            pl.BlockSpec((fd, 64), lambda i: (0, 0)),
            pl.BlockSpec((64, 1), lambda i: (0, 0)),
            pl.BlockSpec((64, 1), lambda i: (0, 0)),
            pl.BlockSpec((1, 1), lambda i: (0, 0)),
        ],
        out_specs=pl.BlockSpec((1, pts_blk), lambda i: (0, i)),
        out_shape=jax.ShapeDtypeStruct((1, n), _F32),
    )(feats_t, W1, b1.reshape(64, 1), W2, b2.reshape(1, 1))


def _row_loss(op_bl, gt_o):
    B, L = gt_o.shape
    rb = 256
    half = L // 2

    def body(o_ref, g_ref, m_ref):
        d = jnp.abs(o_ref[...] - g_ref[...])
        col = lax.broadcasted_iota(_I32, (rb, L), 1)
        w = jnp.where(col < half, _F32(4.0 / 3.0), _F32(2.0 / 3.0))
        m_ref[...] = jnp.mean(d * w, axis=1, keepdims=True)

    return pl.pallas_call(
        body,
        grid=(B // rb,),
        in_specs=[
            pl.BlockSpec((rb, L), lambda i: (i, 0)),
            pl.BlockSpec((rb, L), lambda i: (i, 0)),
        ],
        out_specs=pl.BlockSpec((rb, 1), lambda i: (i, 0)),
        out_shape=jax.ShapeDtypeStruct((B, 1), _F32),
    )(op_bl, gt_o)


def _final_loss(cf_feats_t, m, W1, b1, W2, b2):
    fd, B = cf_feats_t.shape
    dn1 = (((0,), (0,)), ((), ()))

    def body(f_ref, m_ref, w1_ref, b1_ref, w2_ref, b2_ref, o_ref):
        h = lax.dot_general(w1_ref[...], f_ref[...], dn1,
                            preferred_element_type=_F32) + b1_ref[...]
        h = jnp.maximum(h, 0.0)
        conf = lax.dot_general(w2_ref[...], h, dn1,
                               preferred_element_type=_F32) + b2_ref[0, 0]
        lv = jnp.exp(-conf) * m_ref[...] + conf
        o_ref[...] = jnp.mean(lv).reshape(1, 1)

    return pl.pallas_call(
        body,
        in_specs=[
            pl.BlockSpec((fd, B), lambda: (0, 0)),
            pl.BlockSpec((1, B), lambda: (0, 0)),
            pl.BlockSpec((fd, 64), lambda: (0, 0)),
            pl.BlockSpec((64, 1), lambda: (0, 0)),
            pl.BlockSpec((64, 1), lambda: (0, 0)),
            pl.BlockSpec((1, 1), lambda: (0, 0)),
        ],
        out_specs=pl.BlockSpec((1, 1), lambda: (0, 0)),
        out_shape=jax.ShapeDtypeStruct((1, 1), _F32),
    )(cf_feats_t, m, W1, b1.reshape(64, 1), W2, b2.reshape(1, 1))


def kernel(line, line_points, gt_o, op_table, op_W1, op_b1, op_W2, op_b2,
           cf_table, cf_W1, cf_b1, cf_W2, cf_b2):
    B, L, _ = line_points.shape
    n = B * L

    pts = line_points.reshape(n, 3).T
    ln = line.T

    enc3 = _make_encode(3, n, 512)
    enc2 = _make_encode(2, B, B // _NW)
    op_feats = enc3(pts[0], pts[1], pts[2], op_table.reshape(-1))
    cf_feats = enc2(ln[0], ln[1], cf_table.reshape(-1))

    opacity = _mlp_t(op_feats, op_W1, op_b1, op_W2, op_b2, 16000)
    m = _row_loss(opacity.reshape(B, L), gt_o)
    out = _final_loss(cf_feats, m.reshape(1, B), cf_W1, cf_b1, cf_W2, cf_b2)
    return out.reshape(())


# speedup vs baseline: 4.0058x; 3.6208x over previous
import functools
import math

import jax
import jax.numpy as jnp
from jax import lax
from jax.experimental import pallas as pl
from jax.experimental.pallas import tpu as pltpu
from jax.experimental.pallas import tpu_sc as plsc

_NUM_LEVELS = 12
_BASE_RES = 16
_DESIRED_RES = 8192
_HASHMAP_SIZE = 2 ** 21
_PER_LEVEL_SCALE = 2.0 ** (math.log2(_DESIRED_RES / _BASE_RES) / (_NUM_LEVELS - 1))
_PRIMES = (1, 2654435761, 805459861)

_F32 = jnp.float32
_I32 = jnp.int32
_NW = 32


def _level_meta(input_dim):
    metas = []
    offset = 0
    for l in range(_NUM_LEVELS):
        scale = _BASE_RES * (_PER_LEVEL_SCALE ** l) - 1.0
        res = int(math.ceil(scale)) + 1
        params = min(_HASHMAP_SIZE, (res + 1) ** input_dim)
        params = int(math.ceil(params / 8) * 8)
        dense = (res + 1) ** input_dim <= _HASHMAP_SIZE
        metas.append((scale, res, params, offset, dense))
        offset += params
    return metas


def _as_i32(v):
    v = v & 0xFFFFFFFF
    return v - (1 << 32) if v >= (1 << 31) else v


def _make_encode(dim, n_pts, chunk):
    metas = _level_meta(dim)
    ncorner = 2 ** dim
    npw = n_pts // _NW
    nch = npw // chunk
    C = chunk
    grp = C // 16
    nrow = ncorner * C
    mesh = plsc.VectorSubcoreMesh(core_axis_name="c", subcore_axis_name="s")

    @functools.partial(
        pl.kernel,
        mesh=mesh,
        compiler_params=pltpu.CompilerParams(
            needs_layout_passes=False, use_tc_tiling_on_sc=False),
        out_type=jax.ShapeDtypeStruct((2 * _NUM_LEVELS, n_pts), _F32),
        scratch_types=(
            [pltpu.VMEM((C,), _F32) for _ in range(dim)]
            + [
                pltpu.VMEM((2, nrow), _I32),
                pltpu.VMEM((2, ncorner, C), _F32),
                pltpu.VMEM((2, nrow), _I32),
                pltpu.VMEM((2 * _NUM_LEVELS, C), _F32),
                pltpu.SemaphoreType.DMA,
                pltpu.SemaphoreType.DMA,
            ]
        ),
    )
    def enc(*refs):
        coords = refs[:dim]
        table = refs[dim]
        out = refs[dim + 1]
        ubs = refs[dim + 2 : 2 * dim + 2]
        idxb, wb, gb, featsb, sem0, sem1 = refs[2 * dim + 2 :]
        sems = (sem0, sem1)

        wid = lax.axis_index("s") * 2 + lax.axis_index("c")
        base = wid * npw

        def phase_a(buf, scale, res, params, offset, dense):
            def pa_body(g, c2):
                s = g * 16
                pis = []
                fr = []
                for d in range(dim):
                    p = ubs[d][pl.ds(s, 16)] * _F32(scale) + _F32(0.5)
                    pi = p.astype(_I32)
                    pis.append(pi)
                    fr.append(p - pi.astype(_F32))
                a = []
                if dense:
                    strides = [1, res + 1, (res + 1) ** 2]
                    lo = pis[0] + offset
                    a.append((lo, lo + 1))
                    for d in range(1, dim):
                        lo = pis[d] * strides[d]
                        a.append((lo, lo + strides[d]))
                else:
                    a.append((pis[0], pis[0] + 1))
                    for d in range(1, dim):
                        pr = _as_i32(_PRIMES[d])
                        lo = pis[d] * pr
                        a.append((lo, lo + pr))
                w01 = [(_F32(1.0) - fr[d], fr[d]) for d in range(dim)]
                for c in range(ncorner):
                    bits = [(c >> d) & 1 for d in range(dim)]
                    if dense:
                        idx = a[0][bits[0]]
                        for d in range(1, dim):
                            idx = idx + a[d][bits[d]]
                    else:
                        h = a[0][bits[0]]
                        for d in range(1, dim):
                            h = lax.bitwise_xor(h, a[d][bits[d]])
                        idx = lax.bitwise_and(h, params - 1) + offset
                    idxb[buf, pl.ds(c * C + s, 16)] = idx
                    w = w01[0][bits[0]]
                    for d in range(1, dim):
                        w = w * w01[d][bits[d]]
                    wb[buf, c, pl.ds(s, 16)] = w
                return c2

            lax.fori_loop(0, grp, pa_body, 0)

        def phase_c(buf, lvl):
            def pc_body(g, c2):
                s = g * 16
                himask = jnp.full((16,), _as_i32(0xFFFF0000), _I32)
                acc0 = jnp.zeros((16,), _F32)
                acc1 = jnp.zeros((16,), _F32)
                for c in range(ncorner):
                    wv = wb[buf, c, pl.ds(s, 16)]
                    gv = gb[buf, pl.ds(c * C + s, 16)]
                    f0 = plsc.bitcast(lax.shift_left(gv, 16), _F32)
                    f1 = plsc.bitcast(lax.bitwise_and(gv, himask), _F32)
                    acc0 = acc0 + wv * f0
                    acc1 = acc1 + wv * f1
                featsb[2 * lvl, pl.ds(s, 16)] = acc0
                featsb[2 * lvl + 1, pl.ds(s, 16)] = acc1
                return c2

            lax.fori_loop(0, grp, pc_body, 0)

        def chunk_body(ch, carry):
            p0 = base + ch * C
            for d in range(dim):
                pltpu.sync_copy(coords[d].at[pl.ds(p0, C)], ubs[d])

            def u_body(g, c2):
                s = g * 16
                for d in range(dim):
                    x = ubs[d][pl.ds(s, 16)]
                    ubs[d][pl.ds(s, 16)] = (x + 1.0) * 0.5
                return c2

            lax.fori_loop(0, grp, u_body, 0)

            cps = [None, None]
            for lvl, (scale, res, params, offset, dense) in enumerate(metas):
                buf = lvl % 2
                phase_a(buf, scale, res, params, offset, dense)
                cps[buf] = pltpu.async_copy(
                    table.at[idxb.at[buf]], gb.at[buf], sems[buf])
                if lvl > 0:
                    cps[1 - buf].wait()
                    phase_c(1 - buf, lvl - 1)
            cps[1].wait()
            phase_c(1, _NUM_LEVELS - 1)

            pltpu.sync_copy(featsb, out.at[:, pl.ds(p0, C)])
            return carry

        lax.fori_loop(0, nch, chunk_body, 0)

    return enc


def _mlp_t(feats_t, W1, b1, W2, b2, pts_blk):
    fd, n = feats_t.shape
    grid = n // pts_blk
    dn1 = (((0,), (0,)), ((), ()))

    def body(f_ref, w1_ref, b1_ref, w2_ref, b2_ref, o_ref):
        f = f_ref[...]
        h = lax.dot_general(w1_ref[...], f, dn1,
                            preferred_element_type=_F32) + b1_ref[...]
        h = jnp.maximum(h, 0.0)
        o_ref[...] = lax.dot_general(w2_ref[...], h, dn1,
                                     preferred_element_type=_F32) + b2_ref[0, 0]

    return pl.pallas_call(
        body,
        grid=(grid,),
        in_specs=[
            pl.BlockSpec((fd, pts_blk), lambda i: (0, i)),
            pl.BlockSpec((fd, 64), lambda i: (0, 0)),
            pl.BlockSpec((64, 1), lambda i: (0, 0)),
            pl.BlockSpec((64, 1), lambda i: (0, 0)),
            pl.BlockSpec((1, 1), lambda i: (0, 0)),
        ],
        out_specs=pl.BlockSpec((1, pts_blk), lambda i: (0, i)),
        out_shape=jax.ShapeDtypeStruct((1, n), _F32),
    )(feats_t, W1, b1.reshape(64, 1), W2, b2.reshape(1, 1))


def _row_loss(op_bl, gt_o):
    B, L = gt_o.shape
    rb = 256
    half = L // 2

    def body(o_ref, g_ref, m_ref):
        d = jnp.abs(o_ref[...] - g_ref[...])
        col = lax.broadcasted_iota(_I32, (rb, L), 1)
        w = jnp.where(col < half, _F32(4.0 / 3.0), _F32(2.0 / 3.0))
        m_ref[...] = jnp.mean(d * w, axis=1, keepdims=True)

    return pl.pallas_call(
        body,
        grid=(B // rb,),
        in_specs=[
            pl.BlockSpec((rb, L), lambda i: (i, 0)),
            pl.BlockSpec((rb, L), lambda i: (i, 0)),
        ],
        out_specs=pl.BlockSpec((rb, 1), lambda i: (i, 0)),
        out_shape=jax.ShapeDtypeStruct((B, 1), _F32),
    )(op_bl, gt_o)


def _final_loss(cf_feats_t, m, W1, b1, W2, b2):
    fd, B = cf_feats_t.shape
    dn1 = (((0,), (0,)), ((), ()))

    def body(f_ref, m_ref, w1_ref, b1_ref, w2_ref, b2_ref, o_ref):
        h = lax.dot_general(w1_ref[...], f_ref[...], dn1,
                            preferred_element_type=_F32) + b1_ref[...]
        h = jnp.maximum(h, 0.0)
        conf = lax.dot_general(w2_ref[...], h, dn1,
                               preferred_element_type=_F32) + b2_ref[0, 0]
        lv = jnp.exp(-conf) * m_ref[...] + conf
        o_ref[...] = jnp.mean(lv).reshape(1, 1)

    return pl.pallas_call(
        body,
        in_specs=[
            pl.BlockSpec((fd, B), lambda: (0, 0)),
            pl.BlockSpec((1, B), lambda: (0, 0)),
            pl.BlockSpec((fd, 64), lambda: (0, 0)),
            pl.BlockSpec((64, 1), lambda: (0, 0)),
            pl.BlockSpec((64, 1), lambda: (0, 0)),
            pl.BlockSpec((1, 1), lambda: (0, 0)),
        ],
        out_specs=pl.BlockSpec((1, 1), lambda: (0, 0)),
        out_shape=jax.ShapeDtypeStruct((1, 1), _F32),
    )(cf_feats_t, m, W1, b1.reshape(64, 1), W2, b2.reshape(1, 1))


def kernel(line, line_points, gt_o, op_table, op_W1, op_b1, op_W2, op_b2,
           cf_table, cf_W1, cf_b1, cf_W2, cf_b2):
    B, L, _ = line_points.shape
    n = B * L

    pts = line_points.reshape(n, 3).T
    ln = line.T

    enc3 = _make_encode(3, n, 512)
    enc2 = _make_encode(2, B, B // _NW)
    tb3 = lax.bitcast_convert_type(op_table.astype(jnp.bfloat16), _I32)
    tb2 = lax.bitcast_convert_type(cf_table.astype(jnp.bfloat16), _I32)
    op_feats = enc3(pts[0], pts[1], pts[2], tb3)
    cf_feats = enc2(ln[0], ln[1], tb2)

    opacity = _mlp_t(op_feats, op_W1, op_b1, op_W2, op_b2, 16000)
    m = _row_loss(opacity.reshape(B, L), gt_o)
    out = _final_loss(cf_feats, m.reshape(1, B), cf_W1, cf_b1, cf_W2, cf_b2)
    return out.reshape(())
